# baseline (speedup 1.0000x reference)
import functools

import jax
import jax.numpy as jnp
from jax.experimental import pallas as pl
from jax.experimental.pallas import tpu as pltpu

NUM_TOKENS = 8192
EMBED_DIM = 2048
NUM_EXPERTS = 16
TOP_K = 2
BLOCK_N = 2048
NSTEPS = NUM_TOKENS // BLOCK_N


def _gate_body(x_ref, wt_ref, idx_ref, wgt_ref, lbuf):
    i = pl.program_id(0)

    lprev = lbuf[(i + 1) % 2]
    e = jnp.exp(lprev)
    s = jnp.sum(e, axis=-1, keepdims=True)
    colsf = jax.lax.broadcasted_iota(jnp.int32, e.shape, 1).astype(jnp.float32)
    m1 = jnp.max(e, axis=-1, keepdims=True)
    i1 = jnp.min(jnp.where(e == m1, colsf, float(NUM_EXPERTS)),
                 axis=-1, keepdims=True)
    em = jnp.where(colsf == i1, -1.0, e)
    m2 = jnp.max(em, axis=-1, keepdims=True)
    i2 = jnp.min(jnp.where(em == m2, colsf, float(NUM_EXPERTS)),
                 axis=-1, keepdims=True)
    idx_ref[...] = jnp.concatenate([i1, i2], axis=1).astype(jnp.int32)
    wgt_ref[...] = jnp.concatenate([m1, m2], axis=1) / s

    lbuf[i % 2] = jnp.dot(x_ref[...], wt_ref[...],
                          preferred_element_type=jnp.float32)


@functools.partial(jax.jit, static_argnames=())
def kernel(hidden_states, weight):
    n, d = hidden_states.shape
    wt = weight.T
    idx, wgt = pl.pallas_call(
        _gate_body,
        grid=(NSTEPS + 1,),
        in_specs=[
            pl.BlockSpec((BLOCK_N, d),
                         lambda i: (jnp.minimum(i, NSTEPS - 1), 0)),
            pl.BlockSpec((d, NUM_EXPERTS), lambda i: (0, 0)),
        ],
        out_specs=[
            pl.BlockSpec((BLOCK_N, TOP_K),
                         lambda i: (jnp.maximum(i - 1, 0), 0)),
            pl.BlockSpec((BLOCK_N, TOP_K),
                         lambda i: (jnp.maximum(i - 1, 0), 0)),
        ],
        out_shape=[
            jax.ShapeDtypeStruct((n, TOP_K), jnp.int32),
            jax.ShapeDtypeStruct((n, TOP_K), jnp.float32),
        ],
        scratch_shapes=[pltpu.VMEM((2, BLOCK_N, NUM_EXPERTS), jnp.float32)],
        compiler_params=pltpu.CompilerParams(
            dimension_semantics=("arbitrary",),
        ),
    )(hidden_states, wt)
    row_idx = jnp.arange(n * TOP_K, dtype=jnp.int32).reshape(TOP_K, n).T
    return idx, wgt, row_idx


# speedup vs baseline: 1.0745x; 1.0745x over previous
import functools

import jax
import jax.numpy as jnp
from jax.experimental import pallas as pl
from jax.experimental.pallas import tpu as pltpu

NUM_TOKENS = 8192
EMBED_DIM = 2048
NUM_EXPERTS = 16
TOP_K = 2
BLOCK_N = 1024
NSTEPS = NUM_TOKENS // BLOCK_N


def _gate_body(x_ref, wt_ref, idx_ref, wgt_ref, lbuf):
    i = pl.program_id(0)

    lprev = lbuf[(i + 1) % 2]
    e = jnp.exp(lprev)
    s = jnp.sum(e, axis=-1, keepdims=True)
    colsf = jax.lax.broadcasted_iota(jnp.int32, e.shape, 1).astype(jnp.float32)
    m1 = jnp.max(e, axis=-1, keepdims=True)
    i1 = jnp.min(jnp.where(e == m1, colsf, float(NUM_EXPERTS)),
                 axis=-1, keepdims=True)
    em = jnp.where(colsf == i1, -1.0, e)
    m2 = jnp.max(em, axis=-1, keepdims=True)
    i2 = jnp.min(jnp.where(em == m2, colsf, float(NUM_EXPERTS)),
                 axis=-1, keepdims=True)
    idx_ref[...] = jnp.concatenate([i1, i2], axis=1).astype(jnp.int32)
    wgt_ref[...] = jnp.concatenate([m1, m2], axis=1) / s

    lbuf[i % 2] = jnp.dot(x_ref[...], wt_ref[...],
                          preferred_element_type=jnp.float32)


@functools.partial(jax.jit, static_argnames=())
def kernel(hidden_states, weight):
    n, d = hidden_states.shape
    wt = weight.T
    idx, wgt = pl.pallas_call(
        _gate_body,
        grid=(NSTEPS + 1,),
        in_specs=[
            pl.BlockSpec((BLOCK_N, d),
                         lambda i: (jnp.minimum(i, NSTEPS - 1), 0)),
            pl.BlockSpec((d, NUM_EXPERTS), lambda i: (0, 0)),
        ],
        out_specs=[
            pl.BlockSpec((BLOCK_N, TOP_K),
                         lambda i: (jnp.maximum(i - 1, 0), 0)),
            pl.BlockSpec((BLOCK_N, TOP_K),
                         lambda i: (jnp.maximum(i - 1, 0), 0)),
        ],
        out_shape=[
            jax.ShapeDtypeStruct((n, TOP_K), jnp.int32),
            jax.ShapeDtypeStruct((n, TOP_K), jnp.float32),
        ],
        scratch_shapes=[pltpu.VMEM((2, BLOCK_N, NUM_EXPERTS), jnp.float32)],
        compiler_params=pltpu.CompilerParams(
            dimension_semantics=("arbitrary",),
        ),
    )(hidden_states, wt)
    row_idx = jnp.arange(n * TOP_K, dtype=jnp.int32).reshape(TOP_K, n).T
    return idx, wgt, row_idx
